# baseline (speedup 1.0000x reference)
import functools

import jax
import jax.numpy as jnp
from jax import lax
from jax.experimental import pallas as pl
from jax.experimental.pallas import tpu as pltpu
from jax.experimental.pallas import tpu_sc as plsc

_KMAX = 0.2
_KMIN = 0.2
_ALPHA = 0.7
_INT_MIN = -2147483648
_L = 16


def _pos_k(k, n):
    if k <= 0:
        return 0
    elif k < 1:
        return int(round(k * n))
    elif k > n:
        return int(n)
    return int(k)


def _key_fwd(i):
    return jnp.where(i >= 0, i, i ^ jnp.int32(0x7FFFFFFF))


def _make_sc_kernel(num_rows, n, k, alpha):
    nw = 32
    rows_w = num_rows // nw
    chunk = _L
    nchunks = rows_w // chunk
    nv = n // _L
    kk = jnp.int32(k)
    imin = jnp.int32(_INT_MIN)

    def body(in_hbm, out_hbm, xbuf, kbuf, obuf):
        wid = lax.axis_index("s") * 2 + lax.axis_index("c")
        base = wid * rows_w

        def chunk_loop(ci, carry):
            row0 = base + ci * chunk
            pltpu.sync_copy(in_hbm.at[pl.ds(row0, chunk)], xbuf)

            def key_loop(j, carry):
                rr = j // nv
                ee = (j % nv) * _L
                iv = lax.bitcast_convert_type(xbuf[rr, pl.ds(ee, _L)],
                                              jnp.int32)
                kbuf[rr, pl.ds(ee, _L)] = _key_fwd(iv)
                return carry

            lax.fori_loop(0, chunk * nv, key_loop, 0)

            def splat_total(v):
                lanes = lax.broadcasted_iota(jnp.int32, (_L,), 0)
                dnums = lax.GatherDimensionNumbers(
                    offset_dims=(), collapsed_slice_dims=(0,),
                    start_index_map=(0,))
                for j in (1, 2, 4, 8):
                    idx = (lanes ^ j).reshape(_L, 1)
                    v = v + lax.gather(
                        v, idx, dnums, slice_sizes=(1,),
                        mode=lax.GatherScatterMode.PROMISE_IN_BOUNDS)
                return v

            def row_loop(rr, carry):
                def count_pass(vc_hi, vd_lo):
                    def scan(e, acc):
                        ah, al = acc
                        kv = kbuf[rr, pl.ds(e * _L, _L)]
                        ah = ah + jnp.where(kv >= vc_hi, 1, 0)
                        al = al + jnp.where(kv <= vd_lo, 1, 0)
                        return ah, al

                    z = jnp.zeros((_L,), jnp.int32)
                    ah, al = lax.fori_loop(0, nv, scan, (z, z))
                    return splat_total(ah), splat_total(al)

                zv = jnp.zeros((_L,), jnp.int32)
                kkv = jnp.full((_L,), kk, jnp.int32)
                iminv = jnp.full((_L,), imin, jnp.int32)
                cnt0h, cnt0l = count_pass(zv, ~zv)
                ph0 = jnp.where(cnt0h >= kkv, zv, iminv)
                pl0 = jnp.where(cnt0l >= kkv, zv, iminv)

                def bit_step(t, carry2):
                    ph, plo = carry2
                    bitv = jnp.full(
                        (_L,),
                        lax.shift_left(jnp.int32(1), jnp.int32(30) - t),
                        jnp.int32)
                    ch = ph + bitv
                    cl = plo + bitv
                    ch_cnt, cl_cnt = count_pass(ch, ~cl)
                    ph = jnp.where(ch_cnt >= kkv, ch, ph)
                    plo = jnp.where(cl_cnt >= kkv, cl, plo)
                    return ph, plo

                ph, plo = lax.fori_loop(0, 31, bit_step, (ph0, pl0))

                vth = lax.bitcast_convert_type(_key_fwd(ph), jnp.float32)
                vtl = lax.bitcast_convert_type(_key_fwd(~plo), jnp.float32)

                def relu_scan(e, acc):
                    st, sb = acc
                    xv = xbuf[rr, pl.ds(e * _L, _L)]
                    st = st + jnp.maximum(xv - vth, jnp.float32(0.0))
                    sb = sb + jnp.maximum(vtl - xv, jnp.float32(0.0))
                    return st, sb

                zf = jnp.zeros((_L,), jnp.float32)
                st, sb = lax.fori_loop(0, nv, relu_scan, (zf, zf))
                kf = jnp.float32(k)
                s_top = kf * vth + splat_total(st)
                s_bot = kf * vtl - splat_total(sb)
                val = (s_top + jnp.float32(alpha) * s_bot) * jnp.float32(
                    1.0 / (2.0 * k))
                lanes = lax.broadcasted_iota(jnp.int32, (_L,), 0)
                return jnp.where(lanes == rr, val, carry)

            vres = lax.fori_loop(0, chunk, row_loop,
                                 jnp.zeros((_L,), jnp.float32))
            obuf[pl.ds(ci * chunk, _L)] = vres
            return carry

        lax.fori_loop(0, nchunks, chunk_loop, 0)
        pltpu.sync_copy(obuf, out_hbm.at[pl.ds(base, rows_w)])

    return functools.partial(
        pl.kernel,
        out_type=jax.ShapeDtypeStruct((num_rows,), jnp.float32),
        mesh=plsc.VectorSubcoreMesh(core_axis_name="c", subcore_axis_name="s"),
        scratch_types=[
            pltpu.VMEM((chunk, n), jnp.float32),
            pltpu.VMEM((chunk, n), jnp.int32),
            pltpu.VMEM((rows_w,), jnp.float32),
        ],
    )(body)


def kernel(input):
    b, c, h, w = input.shape
    n = h * w
    kmax = _pos_k(_KMAX, n)
    num_rows = b * c
    flat = input.reshape(num_rows, n)
    out = _make_sc_kernel(num_rows, n, kmax, _ALPHA)(flat)
    return out.reshape(b, c)


# speedup vs baseline: 6.5063x; 6.5063x over previous
import functools

import jax
import jax.numpy as jnp
from jax import lax
from jax.experimental import pallas as pl
from jax.experimental.pallas import tpu as pltpu
from jax.experimental.pallas import tpu_sc as plsc

_KMAX = 0.2
_KMIN = 0.2
_ALPHA = 0.7
_INT_MIN = -2147483648
_L = 16


def _pos_k(k, n):
    if k <= 0:
        return 0
    elif k < 1:
        return int(round(k * n))
    elif k > n:
        return int(n)
    return int(k)


def _key_fwd(i):
    return jnp.where(i >= 0, i, i ^ jnp.int32(0x7FFFFFFF))


def _make_sc_kernel(num_rows, n, k, alpha):
    nw = 32
    rows_w = num_rows // nw
    chunk = _L
    nchunks = rows_w // chunk
    nv = n // _L
    kk = jnp.int32(k)
    imin = jnp.int32(_INT_MIN)

    def body(in_hbm, out_hbm, xbuf, kbuf, obuf):
        wid = lax.axis_index("s") * 2 + lax.axis_index("c")
        base = wid * rows_w

        def chunk_loop(ci, carry):
            row0 = base + ci * chunk
            pltpu.sync_copy(in_hbm.at[pl.ds(row0, chunk)], xbuf)

            def key_loop(j, carry):
                rr = j // nv
                ee = (j % nv) * _L
                iv = lax.bitcast_convert_type(xbuf[rr, pl.ds(ee, _L)],
                                              jnp.int32)
                kbuf[rr, pl.ds(ee, _L)] = _key_fwd(iv)
                return carry

            lax.fori_loop(0, chunk * nv, key_loop, 0)

            def splat_total(v):
                lanes = lax.broadcasted_iota(jnp.int32, (_L,), 0)
                dnums = lax.GatherDimensionNumbers(
                    offset_dims=(), collapsed_slice_dims=(0,),
                    start_index_map=(0,))
                for j in (1, 2, 4, 8):
                    idx = (lanes ^ j).reshape(_L, 1)
                    v = v + lax.gather(
                        v, idx, dnums, slice_sizes=(1,),
                        mode=lax.GatherScatterMode.PROMISE_IN_BOUNDS)
                return v

            def row_loop(rr, carry):
                def count_pass(vc_hi, vd_lo):
                    def scan(e, acc):
                        ah, al = acc
                        kv = kbuf[rr, pl.ds(e * _L, _L)]
                        ah = ah + jnp.where(kv >= vc_hi, 1, 0)
                        al = al + jnp.where(kv <= vd_lo, 1, 0)
                        return ah, al

                    z = jnp.zeros((_L,), jnp.int32)
                    ah, al = lax.fori_loop(0, nv, scan, (z, z))
                    return splat_total(ah), splat_total(al)

                zv = jnp.zeros((_L,), jnp.int32)
                kkv = jnp.full((_L,), kk, jnp.int32)
                iminv = jnp.full((_L,), imin, jnp.int32)
                cnt0h, cnt0l = count_pass(zv, ~zv)
                ph0 = jnp.where(cnt0h >= kkv, zv, iminv)
                pl0 = jnp.where(cnt0l >= kkv, zv, iminv)

                def bit_step(t, carry2):
                    ph, plo = carry2
                    bitv = jnp.full(
                        (_L,),
                        lax.shift_left(jnp.int32(1), jnp.int32(30) - t),
                        jnp.int32)
                    ch = ph + bitv
                    cl = plo + bitv
                    ch_cnt, cl_cnt = count_pass(ch, ~cl)
                    ph = jnp.where(ch_cnt >= kkv, ch, ph)
                    plo = jnp.where(cl_cnt >= kkv, cl, plo)
                    return ph, plo

                ph, plo = lax.fori_loop(0, 31, bit_step, (ph0, pl0))

                vth = lax.bitcast_convert_type(_key_fwd(ph), jnp.float32)
                vtl = lax.bitcast_convert_type(_key_fwd(~plo), jnp.float32)

                def relu_scan(e, acc):
                    st, sb = acc
                    xv = xbuf[rr, pl.ds(e * _L, _L)]
                    st = st + jnp.maximum(xv - vth, jnp.float32(0.0))
                    sb = sb + jnp.maximum(vtl - xv, jnp.float32(0.0))
                    return st, sb

                zf = jnp.zeros((_L,), jnp.float32)
                st, sb = lax.fori_loop(0, nv, relu_scan, (zf, zf))
                kf = jnp.float32(k)
                s_top = kf * vth + splat_total(st)
                s_bot = kf * vtl - splat_total(sb)
                val = (s_top + jnp.float32(alpha) * s_bot) * jnp.float32(
                    1.0 / (2.0 * k))
                lanes = lax.broadcasted_iota(jnp.int32, (_L,), 0)
                return jnp.where(lanes == rr, val, carry)

            vres = lax.fori_loop(0, chunk, row_loop,
                                 jnp.zeros((_L,), jnp.float32))
            obuf[pl.ds(ci * chunk, _L)] = vres
            return carry

        lax.fori_loop(0, nchunks, chunk_loop, 0)
        pltpu.sync_copy(obuf, out_hbm.at[pl.ds(base, rows_w)])

    return functools.partial(
        pl.kernel,
        out_type=jax.ShapeDtypeStruct((num_rows,), jnp.float32),
        mesh=plsc.VectorSubcoreMesh(core_axis_name="c", subcore_axis_name="s"),
        scratch_types=[
            pltpu.VMEM((chunk, n), jnp.float32),
            pltpu.VMEM((chunk, n), jnp.int32),
            pltpu.VMEM((rows_w,), jnp.float32),
        ],
    )(body)


def _tc_body(k, alpha, x_ref, o_ref):
    x = x_ref[...]
    xt = x.T
    ikey = _key_fwd(jax.lax.bitcast_convert_type(xt, jnp.int32))
    kk = jnp.int32(k)
    r = x.shape[0]
    imin = jnp.int32(_INT_MIN)

    def counts(c_hi, c_lo):
        v = (jnp.where(ikey >= c_hi, 1, 0)
             + jnp.where(ikey <= ~c_lo, 65536, 0))
        s = jnp.sum(v, axis=0, keepdims=True)
        return s & jnp.int32(0xFFFF), jax.lax.shift_right_logical(
            s, jnp.int32(16))

    zero = jnp.zeros((1, r), jnp.int32)
    cnt0_hi, cnt0_lo = counts(zero, zero)
    init_hi = jnp.where(cnt0_hi >= kk, jnp.int32(0), imin)
    init_lo = jnp.where(cnt0_lo >= kk, jnp.int32(0), imin)

    def bit_body(t, carry):
        p_hi, p_lo = carry
        bit = jax.lax.shift_left(jnp.int32(1), jnp.int32(30) - t)
        c_hi = p_hi + bit
        c_lo = p_lo + bit
        cnt_hi, cnt_lo = counts(c_hi, c_lo)
        p_hi = jnp.where(cnt_hi >= kk, c_hi, p_hi)
        p_lo = jnp.where(cnt_lo >= kk, c_lo, p_lo)
        return p_hi, p_lo

    p_hi, p_lo = jax.lax.fori_loop(0, 31, bit_body, (init_hi, init_lo))

    def key_to_f32(kv):
        return jax.lax.bitcast_convert_type(_key_fwd(kv), jnp.float32)

    t_hi = key_to_f32(p_hi)
    t_lo = key_to_f32(~p_lo)
    kf = jnp.float32(k)
    s_top = kf * t_hi + jnp.sum(jnp.maximum(xt - t_hi, 0.0), axis=0,
                                keepdims=True)
    s_bot = kf * t_lo - jnp.sum(jnp.maximum(t_lo - xt, 0.0), axis=0,
                                keepdims=True)
    out = (s_top + jnp.float32(alpha) * s_bot) * jnp.float32(1.0 / (2.0 * k))
    o_ref[...] = out.reshape(1, 1, r)


def _tc_call(flat, k, alpha):
    num_rows, n = flat.shape
    r = 256
    assert num_rows % r == 0
    out = pl.pallas_call(
        functools.partial(_tc_body, k, alpha),
        grid=(num_rows // r,),
        in_specs=[pl.BlockSpec((r, n), lambda i: (i, 0))],
        out_specs=pl.BlockSpec((1, 1, r), lambda i: (i, 0, 0)),
        out_shape=jax.ShapeDtypeStruct((num_rows // r, 1, r), jnp.float32),
    )(flat)
    return out.reshape(num_rows)


_SC_ROWS = 6144


def kernel(input):
    b, c, h, w = input.shape
    n = h * w
    kmax = _pos_k(_KMAX, n)
    num_rows = b * c
    flat = input.reshape(num_rows, n)
    out_sc = _make_sc_kernel(_SC_ROWS, n, kmax, _ALPHA)(flat[:_SC_ROWS])
    out_tc = _tc_call(flat[_SC_ROWS:], kmax, _ALPHA)
    return jnp.concatenate([out_sc, out_tc]).reshape(b, c)
